# baseline (speedup 1.0000x reference)
import jax
import jax.numpy as jnp
from jax.experimental import pallas as pl
from jax.experimental.pallas import tpu as pltpu

_NE = 8
_D = 768
_F = 1024
_T = 64
_H = _F // 2
_ALPHA = 1.702
_LIMIT = 7.0


def _swiglu(x, w1, b1g, b1u):
    g = jax.lax.dot_general(x, w1[:, :_D], (((1,), (1,)), ((), ())),
                            preferred_element_type=jnp.float32)
    u = jax.lax.dot_general(x, w1[:, _D:], (((1,), (1,)), ((), ())),
                            preferred_element_type=jnp.float32)
    g = jnp.minimum(g + b1g, _LIMIT)
    u = jnp.clip(u + b1u, -_LIMIT, _LIMIT)
    return g * (1.0 / (1.0 + jnp.exp(-_ALPHA * g))) * (u + 1.0)


def _body(x_ref, gw_ref, w1a_ref, w1b_ref, b1g_ref, b1u_ref,
          w2a_ref, w2b_ref, b2_ref, out_ref, s_ref):
    e = pl.program_id(0)

    @pl.when(e == 0)
    def _init():
        gate = jnp.dot(x_ref[...], gw_ref[...], preferred_element_type=jnp.float32)
        idx = jax.lax.broadcasted_iota(jnp.int32, (_T, _NE), 1)
        v1 = jnp.max(gate, axis=1, keepdims=True)
        i1 = jnp.min(jnp.where(gate == v1, idx, _NE), axis=1, keepdims=True)
        masked = jnp.where(idx == i1, -jnp.inf, gate)
        v2 = jnp.max(masked, axis=1, keepdims=True)
        i2 = jnp.min(jnp.where(masked == v2, idx, _NE), axis=1, keepdims=True)
        t = jnp.exp(v2 - v1)
        den = 1.0 + t
        s_ref[...] = (jnp.where(idx == i1, 1.0, 0.0)
                      + jnp.where(idx == i2, t, 0.0)) / den
        out_ref[...] = jnp.zeros_like(out_ref)

    x = x_ref[...]
    act_a = _swiglu(x, w1a_ref[0], b1g_ref[0, 0, :_H], b1u_ref[0, 0, :_H])
    act_b = _swiglu(x, w1b_ref[0], b1g_ref[0, 0, _H:], b1u_ref[0, 0, _H:])
    part = (jax.lax.dot_general(act_a, w2a_ref[0], (((1,), (1,)), ((), ())),
                                preferred_element_type=jnp.float32)
            + jax.lax.dot_general(act_b, w2b_ref[0], (((1,), (1,)), ((), ())),
                                  preferred_element_type=jnp.float32))
    idx = jax.lax.broadcasted_iota(jnp.int32, (_T, _NE), 1)
    s_col = jnp.sum(jnp.where(idx == e, s_ref[...], 0.0), axis=1, keepdims=True)
    out_ref[...] += s_col * (part + b2_ref[0])


def kernel(x, gate_w, dense_1_w, dense_1_b, dense_2_w, dense_2_b):
    B, L, D = x.shape
    x_f = x.reshape(B * L, D)
    w1r = dense_1_w.reshape(_NE, _F, 2 * _D)
    b1g = dense_1_b[:, 0::2].reshape(_NE, 1, _F)
    b1u = dense_1_b[:, 1::2].reshape(_NE, 1, _F)
    b2r = dense_2_b.reshape(_NE, 1, _D)
    out = pl.pallas_call(
        _body,
        grid=(_NE,),
        in_specs=[
            pl.BlockSpec((_T, _D), lambda e: (0, 0)),
            pl.BlockSpec((_D, _NE), lambda e: (0, 0)),
            pl.BlockSpec((1, _H, 2 * _D), lambda e: (e, 0, 0)),
            pl.BlockSpec((1, _H, 2 * _D), lambda e: (e, 1, 0)),
            pl.BlockSpec((1, 1, _F), lambda e: (e, 0, 0)),
            pl.BlockSpec((1, 1, _F), lambda e: (e, 0, 0)),
            pl.BlockSpec((1, _D, _H), lambda e: (e, 0, 0)),
            pl.BlockSpec((1, _D, _H), lambda e: (e, 0, 1)),
            pl.BlockSpec((1, 1, _D), lambda e: (e, 0, 0)),
        ],
        out_specs=pl.BlockSpec((_T, _D), lambda e: (0, 0)),
        out_shape=jax.ShapeDtypeStruct((_T, _D), jnp.float32),
        scratch_shapes=[pltpu.VMEM((_T, _NE), jnp.float32)],
        compiler_params=pltpu.CompilerParams(
            dimension_semantics=("arbitrary",)),
    )(x_f, gate_w, w1r, w1r, b1g, b1u, dense_2_w, dense_2_w, b2r)
    return out.reshape(B, L, D)
